# baseline (speedup 1.0000x reference)
import functools

import jax
import jax.numpy as jnp
from jax import lax
from jax.experimental import pallas as pl
from jax.experimental.pallas import tpu as pltpu
from jax.experimental.pallas import tpu_sc as plsc

_T = 0.05
_BLK = 2048


def _mm_kernel(x_ref, m_ref, o_ref):
    acc = jax.lax.dot_general(
        x_ref[...], m_ref[...],
        dimension_numbers=(((1,), (1,)), ((), ())),
        preferred_element_type=jnp.float32) / _T
    @pl.when(pl.program_id(1) == 0)
    def _():
        o_ref[...] = acc
    @pl.when(pl.program_id(1) != 0)
    def _():
        o_ref[...] += acc


_NC, _NS = 2, 16
_NW = _NC * _NS
_SC_CHUNK = 32
_SC_ROWS = 98304


def _sc_stream_probe(memory):
    rows_per_w = _SC_ROWS // _NW
    n_iters = rows_per_w // _SC_CHUNK
    mesh = plsc.VectorSubcoreMesh(core_axis_name="c", subcore_axis_name="s")

    @functools.partial(
        pl.kernel,
        out_type=jax.ShapeDtypeStruct((_NW, 2048), jnp.float32),
        mesh=mesh,
        scratch_types=[pltpu.VMEM((_SC_CHUNK, 2048), jnp.float32)],
    )
    def k(mem_hbm, out_hbm, buf):
        wid = lax.axis_index("s") * _NC + lax.axis_index("c")
        base = wid * rows_per_w

        def body(i, c):
            pltpu.sync_copy(mem_hbm.at[pl.ds(base + i * _SC_CHUNK, _SC_CHUNK)], buf)
            return c

        lax.fori_loop(0, n_iters, body, 0)
        pltpu.sync_copy(buf.at[0], out_hbm.at[wid])

    return k(memory)


def kernel(x, memory):
    return _tc_kernel(x, memory)


def _tc_kernel(x, memory):
    b, k = x.shape
    n = memory.shape[0]
    kb = k // 2
    return pl.pallas_call(
        _mm_kernel,
        grid=(pl.cdiv(n, _BLK), 2),
        in_specs=[
            pl.BlockSpec((b, kb), lambda i, j: (0, j)),
            pl.BlockSpec((_BLK, kb), lambda i, j: (i, j)),
        ],
        out_specs=pl.BlockSpec((b, _BLK), lambda i, j: (0, i)),
        out_shape=jax.ShapeDtypeStruct((b, n), jnp.float32),
        compiler_params=pltpu.CompilerParams(
            dimension_semantics=("arbitrary", "arbitrary"),
            vmem_limit_bytes=100 * 1024 * 1024),
    )(x, memory)


# speedup vs baseline: 1.0220x; 1.0220x over previous
import jax
import jax.numpy as jnp
from jax.experimental import pallas as pl
from jax.experimental.pallas import tpu as pltpu

_T = 0.05
_BLK = 2048


def _mm_kernel(x_ref, m_ref, o_ref):
    acc = jax.lax.dot_general(
        x_ref[...], m_ref[...],
        dimension_numbers=(((1,), (1,)), ((), ())),
        preferred_element_type=jnp.float32)
    o_ref[...] = acc / _T


def kernel(x, memory):
    b, k = x.shape
    n = memory.shape[0]
    return pl.pallas_call(
        _mm_kernel,
        grid=(pl.cdiv(n, _BLK),),
        in_specs=[
            pl.BlockSpec((b, k), lambda i: (0, 0)),
            pl.BlockSpec((_BLK, k), lambda i: (i, 0)),
        ],
        out_specs=pl.BlockSpec((b, _BLK), lambda i: (0, i)),
        out_shape=jax.ShapeDtypeStruct((b, n), jnp.float32),
        compiler_params=pltpu.CompilerParams(
            dimension_semantics=("arbitrary",),
            vmem_limit_bytes=100 * 1024 * 1024),
    )(x, memory)


# speedup vs baseline: 1.0255x; 1.0034x over previous
import jax
import jax.numpy as jnp
from jax import lax
from jax.experimental import pallas as pl
from jax.experimental.pallas import tpu as pltpu

_T = 0.05
_C = 2048
_NU = 48
_TAIL = ((98304, 0, 512), (98816, 512, 512), (99328, 1024, 512), (99840, 1536, 160))
_TROWS = 1696


def _dot(x, m):
    return jax.lax.dot_general(
        x, m, dimension_numbers=(((1,), (1,)), ((), ())),
        preferred_element_type=jnp.float32) / _T


def _body(x_ref, m_hbm, o_hbm, buf0, buf1, tbuf, ob0, ob1, otb,
          si0, si1, sit, so0, so1, sot):
    x = x_ref[...]
    n_pairs = _NU // 2

    def in_copy(buf, sem, off):
        return pltpu.make_async_copy(m_hbm.at[pl.ds(off, _C)], buf, sem)

    def out_copy(ob, sem, off):
        return pltpu.make_async_copy(ob, o_hbm.at[:, pl.ds(off, _C)], sem)

    in_copy(buf0, si0, 0).start()
    in_copy(buf1, si1, _C).start()

    def loop(j, carry):
        off0 = 2 * j * _C
        off1 = off0 + _C

        in_copy(buf0, si0, off0).wait()

        @pl.when(j > 0)
        def _():
            out_copy(ob0, so0, off0 - 2 * _C).wait()

        ob0[...] = _dot(x, buf0[...])

        @pl.when(j < n_pairs - 1)
        def _():
            in_copy(buf0, si0, off0 + 2 * _C).start()

        @pl.when(j == n_pairs - 1)
        def _():
            for hbm_off, loc_off, cn in _TAIL:
                pltpu.make_async_copy(
                    m_hbm.at[pl.ds(hbm_off, cn)],
                    tbuf.at[pl.ds(loc_off, cn)], sit).start()

        out_copy(ob0, so0, off0).start()

        in_copy(buf1, si1, off1).wait()

        @pl.when(j > 0)
        def _():
            out_copy(ob1, so1, off1 - 2 * _C).wait()

        ob1[...] = _dot(x, buf1[...])

        @pl.when(j < n_pairs - 1)
        def _():
            in_copy(buf1, si1, off1 + 2 * _C).start()

        out_copy(ob1, so1, off1).start()
        return carry

    lax.fori_loop(0, n_pairs, loop, 0)

    for t, (hbm_off, loc_off, cn) in enumerate(_TAIL):
        pltpu.make_async_copy(
            m_hbm.at[pl.ds(hbm_off, cn)], tbuf.at[pl.ds(loc_off, cn)], sit).wait()
        otb[:, pl.ds(loc_off, cn)] = _dot(x, tbuf[pl.ds(loc_off, cn), :])
    pltpu.make_async_copy(
        otb, o_hbm.at[:, pl.ds(98304, _TROWS)], sot).start()

    out_copy(ob0, so0, (_NU - 2) * _C).wait()
    out_copy(ob1, so1, (_NU - 1) * _C).wait()
    pltpu.make_async_copy(otb, o_hbm.at[:, pl.ds(98304, _TROWS)], sot).wait()


def kernel(x, memory):
    b, k = x.shape
    n = memory.shape[0]
    return pl.pallas_call(
        _body,
        in_specs=[
            pl.BlockSpec((b, k), lambda: (0, 0)),
            pl.BlockSpec(memory_space=pl.ANY),
        ],
        out_specs=pl.BlockSpec(memory_space=pl.ANY),
        out_shape=jax.ShapeDtypeStruct((b, n), jnp.float32),
        scratch_shapes=[
            pltpu.VMEM((_C, k), jnp.float32),
            pltpu.VMEM((_C, k), jnp.float32),
            pltpu.VMEM((_TROWS, k), jnp.float32),
            pltpu.VMEM((b, _C), jnp.float32),
            pltpu.VMEM((b, _C), jnp.float32),
            pltpu.VMEM((b, _TROWS), jnp.float32),
            pltpu.SemaphoreType.DMA,
            pltpu.SemaphoreType.DMA,
            pltpu.SemaphoreType.DMA,
            pltpu.SemaphoreType.DMA,
            pltpu.SemaphoreType.DMA,
            pltpu.SemaphoreType.DMA,
        ],
        compiler_params=pltpu.CompilerParams(
            vmem_limit_bytes=100 * 1024 * 1024),
    )(x, memory)
